# Initial kernel scaffold; baseline (speedup 1.0000x reference)
import functools

import jax
import jax.numpy as jnp
from jax import lax
from jax.experimental import pallas as pl
from jax.experimental.pallas import tpu as pltpu
from jax.experimental.pallas import tpu_sc as plsc

NC, NS, LANES = 2, 16, 16
NW = NC * NS
B = 80
C = 128


def _sc_mesh():
    return plsc.VectorSubcoreMesh(
        core_axis_name="c", subcore_axis_name="s", num_cores=NC, num_subcores=NS
    )


def _make_sc_deg(Np, EB):
    KB = EB // NW
    SL = Np // NS

    scratch = [
        pltpu.VMEM((KB, B), jnp.int32),
        pltpu.VMEM((KB, B), jnp.float32),
        pltpu.VMEM_SHARED((Np,), jnp.float32),
    ]

    @functools.partial(
        pl.kernel,
        out_type=jax.ShapeDtypeStruct((NC, Np), jnp.float32),
        mesh=_sc_mesh(),
        scratch_types=scratch,
    )
    def deg_kernel(col2d, ew2d, zer, out, cidx, ewv, acc):
        c = lax.axis_index("c")
        s = lax.axis_index("s")
        wid = c * NS + s
        pltpu.sync_copy(col2d.at[pl.ds(wid * KB, KB)], cidx)
        pltpu.sync_copy(ew2d.at[pl.ds(wid * KB, KB)], ewv)
        pltpu.sync_copy(zer, acc.at[pl.ds(s * SL, SL)])
        plsc.subcore_barrier()

        def body(j, carry):
            pltpu.sync_copy(ewv.at[j], acc.at[cidx.at[j]], add=True)
            return carry

        lax.fori_loop(0, KB, body, 0)
        plsc.subcore_barrier()
        pltpu.sync_copy(acc.at[pl.ds(s * SL, SL)], out.at[c, pl.ds(s * SL, SL)])

    return deg_kernel


def _make_sc_agg(n_chunks, Np, EB):
    KB = EB // NW
    SL = Np // NS

    out_type = [
        jax.ShapeDtypeStruct((NC, Np, C), jnp.float32) for _ in range(n_chunks)
    ]
    scratch = [
        pltpu.VMEM((KB, B), jnp.int32),
        pltpu.VMEM((KB, B), jnp.int32),
        pltpu.VMEM((KB, B), jnp.float32),
        pltpu.VMEM((B, C), jnp.float32),
        pltpu.VMEM_SHARED((Np, C), jnp.float32),
        pltpu.SemaphoreType.DMA,
    ]

    @functools.partial(
        pl.kernel, out_type=out_type, mesh=_sc_mesh(), scratch_types=scratch
    )
    def agg_kernel(*args):
        hs = args[:n_chunks]
        row2d, col2d, ew2d, zer = args[n_chunks : n_chunks + 4]
        outs = args[n_chunks + 4 : 2 * n_chunks + 4]
        ridx, cidx, ewv, msgs, acc, sem = args[2 * n_chunks + 4 :]

        c = lax.axis_index("c")
        s = lax.axis_index("s")
        wid = c * NS + s
        pltpu.sync_copy(row2d.at[pl.ds(wid * KB, KB)], ridx)
        pltpu.sync_copy(col2d.at[pl.ds(wid * KB, KB)], cidx)
        pltpu.sync_copy(ew2d.at[pl.ds(wid * KB, KB)], ewv)

        for ck in range(n_chunks):
            pltpu.sync_copy(zer, acc.at[pl.ds(s * SL, SL)])
            plsc.subcore_barrier()

            def batch(j, carry):
                pltpu.async_copy(hs[ck].at[ridx.at[j]], msgs, sem).wait()

                def mul(e, cc):
                    w = ewv[j, e]
                    for v in range(C // LANES):
                        sl = pl.ds(v * LANES, LANES)
                        msgs[e, sl] = msgs[e, sl] * w
                    return cc

                lax.fori_loop(0, B, mul, 0)
                pltpu.sync_copy(msgs, acc.at[cidx.at[j]], add=True)
                return carry

            lax.fori_loop(0, KB, batch, 0)
            plsc.subcore_barrier()
            pltpu.sync_copy(
                acc.at[pl.ds(s * SL, SL)], outs[ck].at[c, pl.ds(s * SL, SL)]
            )
            if ck + 1 < n_chunks:
                plsc.subcore_barrier()

    return agg_kernel


def _tc_scale_matmul(xp, W1, dis_col, n_chunks, R=1024):
    Np = xp.shape[0]
    D = W1.shape[1]
    G = Np // R

    def body(x_ref, w_ref, d_ref, *o_refs):
        h = jnp.dot(
            x_ref[...], w_ref[...],
            preferred_element_type=jnp.float32,
            precision=lax.Precision.HIGHEST,
        )
        hs = d_ref[...] * h
        for ck in range(n_chunks):
            o_refs[ck][...] = hs[:, ck * C : (ck + 1) * C]

    return pl.pallas_call(
        body,
        grid=(G,),
        in_specs=[
            pl.BlockSpec((R, xp.shape[1]), lambda i: (i, 0)),
            pl.BlockSpec((xp.shape[1], D), lambda i: (0, 0)),
            pl.BlockSpec((R, 1), lambda i: (i, 0)),
        ],
        out_specs=[pl.BlockSpec((R, C), lambda i: (i, 0)) for _ in range(n_chunks)],
        out_shape=[
            jax.ShapeDtypeStruct((Np, C), jnp.float32) for _ in range(n_chunks)
        ],
    )(xp, W1, dis_col)


def _tc_layer2(S1, H1c, dis_col, b1r, W2r, a2, R=1024):
    n_chunks = len(S1)
    Np = dis_col.shape[0]
    G = Np // R

    def body(*refs):
        s_refs = refs[:n_chunks]
        h_refs = refs[n_chunks : 2 * n_chunks]
        d_ref, b_ref, w_ref, a_ref, o_ref = refs[2 * n_chunks :]
        d = d_ref[...]
        a = a_ref[0, 0]
        acc = jnp.zeros((d.shape[0], C), jnp.float32)
        for ck in range(n_chunks):
            t = s_refs[ck][0] + s_refs[ck][1] + h_refs[ck][...]
            f = d * t + b_ref[ck][None, :]
            f = jnp.maximum(f, 0.0) + a * jnp.minimum(f, 0.0)
            acc = acc + jnp.dot(
                f, w_ref[ck],
                preferred_element_type=jnp.float32,
                precision=lax.Precision.HIGHEST,
            )
        o_ref[...] = d * acc

    return pl.pallas_call(
        body,
        grid=(G,),
        in_specs=(
            [pl.BlockSpec((NC, R, C), lambda i: (0, i, 0)) for _ in range(n_chunks)]
            + [pl.BlockSpec((R, C), lambda i: (i, 0)) for _ in range(n_chunks)]
            + [
                pl.BlockSpec((R, 1), lambda i: (i, 0)),
                pl.BlockSpec((n_chunks, C), lambda i: (0, 0)),
                pl.BlockSpec((n_chunks, C, C), lambda i: (0, 0, 0)),
                pl.BlockSpec((1, 1), lambda i: (0, 0)),
            ]
        ),
        out_specs=pl.BlockSpec((R, C), lambda i: (i, 0)),
        out_shape=jax.ShapeDtypeStruct((Np, C), jnp.float32),
    )(*S1, *H1c, dis_col, b1r, W2r, a2)


def _tc_final(S2, h2, dis_col, b2, a2, R=1024):
    Np = dis_col.shape[0]
    G = Np // R

    def body(s_ref, h_ref, d_ref, b_ref, a_ref, o_ref):
        d = d_ref[...]
        a = a_ref[0, 0]
        t = s_ref[0] + s_ref[1] + h_ref[...]
        f = d * t + b_ref[0][None, :]
        o_ref[...] = jnp.maximum(f, 0.0) + a * jnp.minimum(f, 0.0)

    return pl.pallas_call(
        body,
        grid=(G,),
        in_specs=[
            pl.BlockSpec((NC, R, C), lambda i: (0, i, 0)),
            pl.BlockSpec((R, C), lambda i: (i, 0)),
            pl.BlockSpec((R, 1), lambda i: (i, 0)),
            pl.BlockSpec((1, C), lambda i: (0, 0)),
            pl.BlockSpec((1, 1), lambda i: (0, 0)),
        ],
        out_specs=pl.BlockSpec((R, C), lambda i: (i, 0)),
        out_shape=jax.ShapeDtypeStruct((Np, C), jnp.float32),
    )(S2, h2, dis_col, b2, a2)


def kernel(x, edge_index, edge_attr, W1, b1, W2, b2, a):
    N, D_IN = x.shape
    E = edge_attr.shape[0]
    H1 = W1.shape[1]
    H2 = W2.shape[1]
    n_ch1 = H1 // C

    Np = ((N + 1023) // 1024) * 1024
    xp = jnp.pad(x, ((0, Np - N), (0, 0)))

    row2d = edge_index[0].reshape(E // B, B)
    col2d = edge_index[1].reshape(E // B, B)
    ew2d = edge_attr.reshape(E // B, B)

    SL = Np // NS
    zer1 = jnp.zeros((SL,), jnp.float32)
    zer2 = jnp.zeros((SL, C), jnp.float32)

    degp = _make_sc_deg(Np, E // B)(col2d, ew2d, zer1)
    deg = degp[0] + degp[1] + 1.0
    dis_col = lax.rsqrt(deg)[:, None]

    H1c = _tc_scale_matmul(xp, W1, dis_col, n_ch1)

    S1 = _make_sc_agg(n_ch1, Np, E // B)(*H1c, row2d, col2d, ew2d, zer2)

    b1r = b1.reshape(n_ch1, C)
    W2r = W2.reshape(n_ch1, C, H2)
    a2 = jnp.asarray(a, jnp.float32).reshape(1, 1)
    h2 = _tc_layer2(S1, H1c, dis_col, b1r, W2r, a2)

    (S2,) = _make_sc_agg(1, Np, E // B)(h2, row2d, col2d, ew2d, zer2)

    feat2 = _tc_final(S2, h2, dis_col, b2.reshape(1, C), a2)
    return feat2[:N]


# speedup vs baseline: 4.7750x; 4.7750x over previous
import functools

import jax
import jax.numpy as jnp
from jax import lax
from jax.experimental import pallas as pl
from jax.experimental.pallas import tpu as pltpu
from jax.experimental.pallas import tpu_sc as plsc

NC, NS, LANES = 2, 16, 16
NW = NC * NS
B = 128
C = 128


def _sc_mesh():
    return plsc.VectorSubcoreMesh(
        core_axis_name="c", subcore_axis_name="s", num_cores=NC, num_subcores=NS
    )


def _make_sc_deg(Np, EB):
    KB = EB // NW
    SL = Np // NS

    scratch = [
        pltpu.VMEM((KB, B), jnp.int32),
        pltpu.VMEM((KB, B), jnp.float32),
        pltpu.VMEM_SHARED((Np,), jnp.float32),
    ]

    @functools.partial(
        pl.kernel,
        out_type=jax.ShapeDtypeStruct((NC * Np,), jnp.float32),
        mesh=_sc_mesh(),
        scratch_types=scratch,
    )
    def deg_kernel(col2d, ew2d, zer, out, cidx, ewv, acc):
        c = lax.axis_index("c")
        s = lax.axis_index("s")
        wid = c * NS + s
        pltpu.sync_copy(col2d.at[pl.ds(wid * KB, KB)], cidx)
        pltpu.sync_copy(ew2d.at[pl.ds(wid * KB, KB)], ewv)
        pltpu.sync_copy(zer, acc.at[pl.ds(s * SL, SL)])
        plsc.subcore_barrier()

        def body(j, carry):
            pltpu.sync_copy(ewv.at[j], acc.at[cidx.at[j]], add=True)
            return carry

        lax.fori_loop(0, KB, body, 0)
        plsc.subcore_barrier()
        pltpu.sync_copy(acc.at[pl.ds(s * SL, SL)], out.at[pl.ds(c * Np + s * SL, SL)])

    return deg_kernel


def _make_sc_agg(n_chunks, Np, EB):
    KB = EB // NW
    SL = Np // NS

    out_type = [
        jax.ShapeDtypeStruct((NC, Np, C), jnp.float32) for _ in range(n_chunks)
    ]
    scratch = [
        pltpu.VMEM((KB, B), jnp.int32),
        pltpu.VMEM((KB, B), jnp.int32),
        pltpu.VMEM((KB, B), jnp.float32),
        pltpu.VMEM((B, C), jnp.float32),
        pltpu.VMEM_SHARED((Np, C), jnp.float32),
        pltpu.SemaphoreType.DMA,
    ]

    @functools.partial(
        pl.kernel, out_type=out_type, mesh=_sc_mesh(), scratch_types=scratch
    )
    def agg_kernel(*args):
        hs = args[:n_chunks]
        row2d, col2d, ew2d, zer = args[n_chunks : n_chunks + 4]
        outs = args[n_chunks + 4 : 2 * n_chunks + 4]
        ridx, cidx, ewv, msgs, acc, sem = args[2 * n_chunks + 4 :]

        c = lax.axis_index("c")
        s = lax.axis_index("s")
        wid = c * NS + s
        pltpu.sync_copy(row2d.at[pl.ds(wid * KB, KB)], ridx)
        pltpu.sync_copy(col2d.at[pl.ds(wid * KB, KB)], cidx)
        pltpu.sync_copy(ew2d.at[pl.ds(wid * KB, KB)], ewv)

        for ck in range(n_chunks):
            pltpu.sync_copy(zer, acc.at[pl.ds(s * SL, SL)])
            plsc.subcore_barrier()

            def batch(j, carry):
                pltpu.async_copy(hs[ck].at[ridx.at[j]], msgs, sem).wait()

                def mul(g, cc):
                    wv = ewv[j, pl.ds(g * LANES, LANES)]
                    for l in range(LANES):
                        w = wv[l]
                        e = g * LANES + l
                        for v in range(C // LANES):
                            sl = pl.ds(v * LANES, LANES)
                            msgs[e, sl] = msgs[e, sl] * w
                    return cc

                lax.fori_loop(0, B // LANES, mul, 0)
                pltpu.sync_copy(msgs, acc.at[cidx.at[j]], add=True)
                return carry

            lax.fori_loop(0, KB, batch, 0)
            plsc.subcore_barrier()
            pltpu.sync_copy(
                acc.at[pl.ds(s * SL, SL)], outs[ck].at[c, pl.ds(s * SL, SL)]
            )
            if ck + 1 < n_chunks:
                plsc.subcore_barrier()

    return agg_kernel


def _tc_scale_matmul(xp, W1, dis_col, n_chunks, R=1024):
    Np = xp.shape[0]
    D = W1.shape[1]
    G = Np // R

    def body(x_ref, w_ref, d_ref, *o_refs):
        h = jnp.dot(
            x_ref[...], w_ref[...],
            preferred_element_type=jnp.float32,
            precision=lax.Precision.HIGHEST,
        )
        hs = d_ref[...] * h
        for ck in range(n_chunks):
            o_refs[ck][...] = hs[:, ck * C : (ck + 1) * C]

    return pl.pallas_call(
        body,
        grid=(G,),
        in_specs=[
            pl.BlockSpec((R, xp.shape[1]), lambda i: (i, 0)),
            pl.BlockSpec((xp.shape[1], D), lambda i: (0, 0)),
            pl.BlockSpec((R, 1), lambda i: (i, 0)),
        ],
        out_specs=[pl.BlockSpec((R, C), lambda i: (i, 0)) for _ in range(n_chunks)],
        out_shape=[
            jax.ShapeDtypeStruct((Np, C), jnp.float32) for _ in range(n_chunks)
        ],
    )(xp, W1, dis_col)


def _tc_layer2(S1, H1c, dis_col, b1r, W2r, a2, R=1024):
    n_chunks = len(S1)
    Np = dis_col.shape[0]
    G = Np // R

    def body(*refs):
        s_refs = refs[:n_chunks]
        h_refs = refs[n_chunks : 2 * n_chunks]
        d_ref, b_ref, w_ref, a_ref, o_ref = refs[2 * n_chunks :]
        d = d_ref[...]
        a = a_ref[0, 0]
        acc = jnp.zeros((d.shape[0], C), jnp.float32)
        for ck in range(n_chunks):
            t = s_refs[ck][0] + s_refs[ck][1] + h_refs[ck][...]
            f = d * t + b_ref[ck][None, :]
            f = jnp.maximum(f, 0.0) + a * jnp.minimum(f, 0.0)
            acc = acc + jnp.dot(
                f, w_ref[ck],
                preferred_element_type=jnp.float32,
                precision=lax.Precision.HIGHEST,
            )
        o_ref[...] = d * acc

    return pl.pallas_call(
        body,
        grid=(G,),
        in_specs=(
            [pl.BlockSpec((NC, R, C), lambda i: (0, i, 0)) for _ in range(n_chunks)]
            + [pl.BlockSpec((R, C), lambda i: (i, 0)) for _ in range(n_chunks)]
            + [
                pl.BlockSpec((R, 1), lambda i: (i, 0)),
                pl.BlockSpec((n_chunks, C), lambda i: (0, 0)),
                pl.BlockSpec((n_chunks, C, C), lambda i: (0, 0, 0)),
                pl.BlockSpec((1, 1), lambda i: (0, 0)),
            ]
        ),
        out_specs=pl.BlockSpec((R, C), lambda i: (i, 0)),
        out_shape=jax.ShapeDtypeStruct((Np, C), jnp.float32),
    )(*S1, *H1c, dis_col, b1r, W2r, a2)


def _tc_final(S2, h2, dis_col, b2, a2, R=1024):
    Np = dis_col.shape[0]
    G = Np // R

    def body(s_ref, h_ref, d_ref, b_ref, a_ref, o_ref):
        d = d_ref[...]
        a = a_ref[0, 0]
        t = s_ref[0] + s_ref[1] + h_ref[...]
        f = d * t + b_ref[0][None, :]
        o_ref[...] = jnp.maximum(f, 0.0) + a * jnp.minimum(f, 0.0)

    return pl.pallas_call(
        body,
        grid=(G,),
        in_specs=[
            pl.BlockSpec((NC, R, C), lambda i: (0, i, 0)),
            pl.BlockSpec((R, C), lambda i: (i, 0)),
            pl.BlockSpec((R, 1), lambda i: (i, 0)),
            pl.BlockSpec((1, C), lambda i: (0, 0)),
            pl.BlockSpec((1, 1), lambda i: (0, 0)),
        ],
        out_specs=pl.BlockSpec((R, C), lambda i: (i, 0)),
        out_shape=jax.ShapeDtypeStruct((Np, C), jnp.float32),
    )(S2, h2, dis_col, b2, a2)


def kernel(x, edge_index, edge_attr, W1, b1, W2, b2, a):
    N, D_IN = x.shape
    E = edge_attr.shape[0]
    H1 = W1.shape[1]
    H2 = W2.shape[1]
    n_ch1 = H1 // C

    Np = ((N + 1023) // 1024) * 1024
    xp = jnp.pad(x, ((0, Np - N), (0, 0)))

    Ep = ((E + NW * 8 * B - 1) // (NW * 8 * B)) * (NW * 8 * B)
    row2d = jnp.pad(edge_index[0], (0, Ep - E)).reshape(Ep // B, B)
    col2d = jnp.pad(edge_index[1], (0, Ep - E)).reshape(Ep // B, B)
    ew2d = jnp.pad(edge_attr, (0, Ep - E)).reshape(Ep // B, B)
    E = Ep

    SL = Np // NS
    zer1 = jnp.zeros((SL,), jnp.float32)
    zer2 = jnp.zeros((SL, C), jnp.float32)

    degp = _make_sc_deg(Np, E // B)(col2d, ew2d, zer1).reshape(NC, Np)
    deg = degp[0] + degp[1] + 1.0
    dis_col = lax.rsqrt(deg)[:, None]

    H1c = _tc_scale_matmul(xp, W1, dis_col, n_ch1)

    S1 = _make_sc_agg(n_ch1, Np, E // B)(*H1c, row2d, col2d, ew2d, zer2)

    b1r = b1.reshape(n_ch1, C)
    W2r = W2.reshape(n_ch1, C, H2)
    a2 = jnp.asarray(a, jnp.float32).reshape(1, 1)
    h2 = _tc_layer2(S1, H1c, dis_col, b1r, W2r, a2)

    (S2,) = _make_sc_agg(1, Np, E // B)(h2, row2d, col2d, ew2d, zer2)

    feat2 = _tc_final(S2, h2, dis_col, b2.reshape(1, C), a2)
    return feat2[:N]
